# baseline (speedup 1.0000x reference)
import jax
import jax.numpy as jnp
from jax import lax
from jax.experimental import pallas as pl
from jax.experimental.pallas import tpu as pltpu
from jax.experimental.pallas import tpu_sc as plsc

N = 16384
DIM = 32
DEPTH = 20
NC = 2
NS = 16
NW = NC * NS
TOK_W = N // NW
T = 64
NCH = TOK_W // T
CH = T * DEPTH
GB = 256
L = 16


def _body(x_hbm, lab_hbm, emb1_hbm, emb2_hbm,
          out_hbm, tgt_hbm,
          idx_v, proj_v, dir_v, lab_v, hvec_v, out_v, tgt_v,
          sem_p, sem_l, sem_g0, sem_g1, sem_o0, sem_o1):
    sem_g = (sem_g0, sem_g1)
    sem_o = (sem_o0, sem_o1)
    wid = lax.axis_index("s") * NC + lax.axis_index("c")
    tok0 = wid * TOK_W
    iota = lax.iota(jnp.int32, L)
    zeros = jnp.zeros((L,), jnp.int32)
    ones = jnp.full((L,), 1, jnp.int32)

    pltpu.sync_copy(x_hbm.at[pl.ds(tok0, TOK_W)], idx_v)
    for j in range(TOK_W // GB):
        pltpu.async_copy(
            emb1_hbm.at[idx_v.at[pl.ds(j * GB, GB)]],
            proj_v.at[pl.ds(j * GB, GB)], sem_p)

    def lab_copy(c):
        return pltpu.make_async_copy(
            lab_hbm.at[pl.ds(tok0 + c * T, T)],
            lab_v.at[pl.ds((c & 3) * T, T)], sem_l)

    def peel_and_fire(c, b):
        for blk in range(T // L):
            tok = (c & 3) * T + blk * L + iota
            dst = b * CH + blk * L * DEPTH + iota * DEPTH
            for k in range(DEPTH):
                v = plsc.load_gather(lab_v, [tok, zeros, zeros,
                                             jnp.full((L,), k, jnp.int32)])
                plsc.store_scatter(dir_v, [dst + k], v)
        for j in range(CH // GB):
            pltpu.async_copy(
                emb2_hbm.at[dir_v.at[pl.ds(b * CH + j * GB, GB)]],
                hvec_v.at[pl.ds(b * CH + j * GB, GB)], sem_g[b])

    def wait_chunk(b):
        for j in range(CH // GB):
            pltpu.make_async_copy(
                emb2_hbm.at[dir_v.at[pl.ds(b * CH + j * GB, GB)]],
                hvec_v.at[pl.ds(b * CH + j * GB, GB)], sem_g[b]).wait()

    for b in range(2):
        lab_copy(jnp.int32(b)).start()
        lab_copy(jnp.int32(b)).wait()
        peel_and_fire(jnp.int32(b), b)
    for j in range(TOK_W // GB):
        pltpu.make_async_copy(
            emb1_hbm.at[idx_v.at[pl.ds(j * GB, GB)]],
            proj_v.at[pl.ds(j * GB, GB)], sem_p).wait()

    def out_copies(c, b):
        return (
            pltpu.make_async_copy(
                out_v.at[pl.ds(b * CH, CH)],
                out_hbm.at[pl.ds(tok0 * DEPTH + c * CH, CH)], sem_o[b]),
            pltpu.make_async_copy(
                tgt_v.at[pl.ds(b * CH, CH)],
                tgt_hbm.at[pl.ds(tok0 * DEPTH + c * CH, CH)], sem_o[b]),
        )

    def compute_chunk(c, b):
        for blk in range(T // L):
            rows = [b * CH + blk * L * DEPTH + iota * DEPTH + k
                    for k in range(DEPTH)]
            tok = (c & 3) * T + blk * L + iota
            rowp = c * T + blk * L + iota

            def dstep(d, accs):
                dv = jnp.full((L,), d, jnp.int32)
                pv = plsc.load_gather(proj_v, [rowp, dv])
                return tuple(
                    accs[k] + pv * plsc.load_gather(hvec_v, [rows[k], dv])
                    for k in range(DEPTH))

            accs = lax.fori_loop(
                0, DIM, dstep, (jnp.zeros((L,), jnp.float32),) * DEPTH)
            for k in range(DEPTH):
                x = accs[k]
                o = 1.0 / (1.0 + jnp.exp(-x))
                labv = plsc.load_gather(lab_v, [tok, zeros, ones,
                                                jnp.full((L,), k, jnp.int32)])
                mi = jnp.where(x >= 0.0, 1, 0)
                t = jnp.where(mi == labv, 1.0, 0.0)
                plsc.store_scatter(out_v, [rows[k]], o)
                plsc.store_scatter(tgt_v, [rows[k]], t)

    def step(i, carry):
        for b in range(2):
            c = 2 * i + b
            wait_chunk(b)

            @pl.when(c + 2 < NCH)
            def _():
                lab_copy(c + 2).start()

            @pl.when(c >= 2)
            def _():
                for cp in out_copies(c - 2, b):
                    cp.wait()

            compute_chunk(c, b)
            for cp in out_copies(c, b):
                cp.start()

            @pl.when(c + 2 < NCH)
            def _():
                lab_copy(c + 2).wait()
                peel_and_fire(c + 2, b)
        return carry

    lax.fori_loop(0, NCH // 2, step, 0)
    for b in range(2):
        for cp in out_copies(NCH - 2 + b, b):
            cp.wait()


V = 1000000
VP = 1000008
DG = 8
TB = 2000
TR = V // 4
TNB = TR // TB


def _tbody(e1_hbm, e2_hbm, r1_hbm, r2_hbm,
           inb_v, outb_v, sem_i0, sem_i1, sem_t0, sem_t1):
    sem_i = (sem_i0, sem_i1)
    sem_t = (sem_t0, sem_t1)
    cid = lax.axis_index("c")
    sid = lax.axis_index("s")
    dg = (sid % 4) * DG
    tr0 = (sid // 4) * TR
    iota = lax.iota(jnp.int32, L)

    def job(src, dst):
        def in_copy(c, b, d):
            return pltpu.make_async_copy(
                src.at[pl.ds((dg + d) * VP + tr0 + c * TB, TB)],
                inb_v.at[pl.ds((b * DG + d) * TB, TB)], sem_i[b])

        def in_copies(c, b):
            for d in range(DG):
                in_copy(c, b, d).start()

        def in_waits(c, b):
            for d in range(DG):
                in_copy(c, b, d).wait()

        def out_copy(c, b):
            return pltpu.make_async_copy(
                outb_v.at[pl.ds(b * TB, TB)],
                dst.at[pl.ds(tr0 + c * TB, TB), pl.ds(dg, DG)], sem_t[b])

        def transpose(b):
            def gstep(g, carry):
                srow = g * L + iota
                drow = b * TB + srow
                for d in range(DG):
                    v = plsc.load_gather(inb_v, [(b * DG + d) * TB + srow])
                    plsc.store_scatter(outb_v, [drow,
                                                jnp.full((L,), d, jnp.int32)],
                                       v)
                return carry

            lax.fori_loop(0, TB // L, gstep, 0)

        for b in range(2):
            in_copies(jnp.int32(b), b)

        def step(i, carry):
            for b in range(2):
                c = 2 * i + b

                @pl.when(c < TNB)
                def _():
                    in_waits(c, b)

                    @pl.when(c >= 2)
                    def _():
                        out_copy(c - 2, b).wait()

                    transpose(b)
                    out_copy(c, b).start()

                    @pl.when(c + 2 < TNB)
                    def _():
                        in_copies(c + 2, b)
            return carry

        lax.fori_loop(0, (TNB + 1) // 2, step, 0)
        out_copy(TNB - 2, 0).wait()
        out_copy(TNB - 1, 1).wait()

    @pl.when(cid == 0)
    def _():
        job(e1_hbm, r1_hbm)

    @pl.when(cid == 1)
    def _():
        job(e2_hbm, r2_hbm)


@jax.jit
def _transpose(e1v, e2v):
    mesh = plsc.VectorSubcoreMesh(core_axis_name="c", subcore_axis_name="s",
                                  num_cores=NC, num_subcores=NS)
    f = pl.kernel(
        _tbody,
        out_type=(jax.ShapeDtypeStruct((V, DIM), jnp.float32),
                  jax.ShapeDtypeStruct((V, DIM), jnp.float32)),
        mesh=mesh,
        compiler_params=pltpu.CompilerParams(needs_layout_passes=False,
                                             use_tc_tiling_on_sc=False),
        scratch_types=[
            pltpu.VMEM((2 * DG * TB,), jnp.float32),
            pltpu.VMEM((2 * TB, DG), jnp.float32),
            pltpu.SemaphoreType.DMA,
            pltpu.SemaphoreType.DMA,
            pltpu.SemaphoreType.DMA,
            pltpu.SemaphoreType.DMA,
        ],
    )
    return f(e1v, e2v)


@jax.jit
def _run(x, label, emb1, emb2):
    mesh = plsc.VectorSubcoreMesh(core_axis_name="c", subcore_axis_name="s",
                                  num_cores=NC, num_subcores=NS)
    f = pl.kernel(
        _body,
        out_type=(jax.ShapeDtypeStruct((N * DEPTH,), jnp.float32),
                  jax.ShapeDtypeStruct((N * DEPTH,), jnp.float32)),
        mesh=mesh,
        compiler_params=pltpu.CompilerParams(needs_layout_passes=False,
                                             use_tc_tiling_on_sc=False),
        scratch_types=[
            pltpu.VMEM((TOK_W,), jnp.int32),
            pltpu.VMEM((TOK_W, DIM), jnp.float32),
            pltpu.VMEM((2 * CH,), jnp.int32),
            pltpu.VMEM((4 * T, 1, 2, DEPTH), jnp.int32),
            pltpu.VMEM((2 * CH, DIM), jnp.float32),
            pltpu.VMEM((2 * CH,), jnp.float32),
            pltpu.VMEM((2 * CH,), jnp.float32),
            pltpu.SemaphoreType.DMA,
            pltpu.SemaphoreType.DMA,
            pltpu.SemaphoreType.DMA,
            pltpu.SemaphoreType.DMA,
            pltpu.SemaphoreType.DMA,
            pltpu.SemaphoreType.DMA,
        ],
    )
    return f(x, label, emb1, emb2)


def kernel(inputs, label, emb1, emb2):
    n, p, _, depth = label.shape
    assert (n, p, depth) == (N, 1, DEPTH) and emb1.shape[1] == DIM
    assert label.dtype == jnp.int32 and inputs.dtype == jnp.int32
    e1v = jnp.pad(emb1.T, ((0, 0), (0, VP - emb1.shape[0]))).reshape(DIM * VP)
    e2v = jnp.pad(emb2.T, ((0, 0), (0, VP - emb2.shape[0]))).reshape(DIM * VP)
    r1, r2 = _transpose(e1v, e2v)
    o, t = _run(inputs, label, r1, r2)
    return o.reshape(N, 1, DEPTH), t.reshape(N, 1, DEPTH)


# speedup vs baseline: 5.1797x; 5.1797x over previous
import jax
import jax.numpy as jnp
from jax import lax
from jax.experimental import pallas as pl
from jax.experimental.pallas import tpu as pltpu
from jax.experimental.pallas import tpu_sc as plsc

N = 16384
DIM = 32
DEPTH = 20
NC = 2
NS = 16
NW = NC * NS
TOK_W = N // NW
T = 64
NCH = TOK_W // T
CH = T * DEPTH
GB = 256
L = 16


def _body(x_hbm, lab_hbm, emb1_hbm, emb2_hbm,
          out_hbm, tgt_hbm,
          idx_v, proj_v, dir_v, lab_v, hvec_v, out_v, tgt_v,
          sem_p, sem_l, sem_g0, sem_g1, sem_o0, sem_o1):
    sem_g = (sem_g0, sem_g1)
    sem_o = (sem_o0, sem_o1)
    wid = lax.axis_index("s") * NC + lax.axis_index("c")
    tok0 = wid * TOK_W
    iota = lax.iota(jnp.int32, L)
    zeros = jnp.zeros((L,), jnp.int32)
    ones = jnp.full((L,), 1, jnp.int32)

    pltpu.sync_copy(x_hbm.at[pl.ds(tok0, TOK_W)], idx_v)
    for j in range(TOK_W // GB):
        pltpu.async_copy(
            emb1_hbm.at[idx_v.at[pl.ds(j * GB, GB)]],
            proj_v.at[pl.ds(j * GB, GB)], sem_p)

    def lab_copy(c):
        return pltpu.make_async_copy(
            lab_hbm.at[pl.ds(tok0 + c * T, T)],
            lab_v.at[pl.ds((c & 3) * T, T)], sem_l)

    def peel_and_fire(c, b):
        for blk in range(T // L):
            tok = (c & 3) * T + blk * L + iota
            dst = b * CH + blk * L * DEPTH + iota * DEPTH
            for k in range(DEPTH):
                v = plsc.load_gather(lab_v, [tok, zeros, zeros,
                                             jnp.full((L,), k, jnp.int32)])
                plsc.store_scatter(dir_v, [dst + k], v)
        for j in range(CH // GB):
            pltpu.async_copy(
                emb2_hbm.at[dir_v.at[pl.ds(b * CH + j * GB, GB)]],
                hvec_v.at[pl.ds(b * CH + j * GB, GB)], sem_g[b])

    def wait_chunk(b):
        for j in range(CH // GB):
            pltpu.make_async_copy(
                emb2_hbm.at[dir_v.at[pl.ds(b * CH + j * GB, GB)]],
                hvec_v.at[pl.ds(b * CH + j * GB, GB)], sem_g[b]).wait()

    for b in range(2):
        lab_copy(jnp.int32(b)).start()
        lab_copy(jnp.int32(b)).wait()
        peel_and_fire(jnp.int32(b), b)
    for j in range(TOK_W // GB):
        pltpu.make_async_copy(
            emb1_hbm.at[idx_v.at[pl.ds(j * GB, GB)]],
            proj_v.at[pl.ds(j * GB, GB)], sem_p).wait()

    def out_copies(c, b):
        return (
            pltpu.make_async_copy(
                out_v.at[pl.ds(b * CH, CH)],
                out_hbm.at[pl.ds(tok0 * DEPTH + c * CH, CH)], sem_o[b]),
            pltpu.make_async_copy(
                tgt_v.at[pl.ds(b * CH, CH)],
                tgt_hbm.at[pl.ds(tok0 * DEPTH + c * CH, CH)], sem_o[b]),
        )

    def compute_chunk(c, b):
        for blk in range(T // L):
            rows = [b * CH + blk * L * DEPTH + iota * DEPTH + k
                    for k in range(DEPTH)]
            tok = (c & 3) * T + blk * L + iota
            rowp = c * T + blk * L + iota

            def dstep(d, accs):
                dv = jnp.full((L,), d, jnp.int32)
                pv = plsc.load_gather(proj_v, [rowp, dv])
                return tuple(
                    accs[k] + pv * plsc.load_gather(hvec_v, [rows[k], dv])
                    for k in range(DEPTH))

            accs = lax.fori_loop(
                0, DIM, dstep, (jnp.zeros((L,), jnp.float32),) * DEPTH)
            for k in range(DEPTH):
                x = accs[k]
                o = 1.0 / (1.0 + jnp.exp(-x))
                labv = plsc.load_gather(lab_v, [tok, zeros, ones,
                                                jnp.full((L,), k, jnp.int32)])
                mi = jnp.where(x >= 0.0, 1, 0)
                t = jnp.where(mi == labv, 1.0, 0.0)
                plsc.store_scatter(out_v, [rows[k]], o)
                plsc.store_scatter(tgt_v, [rows[k]], t)

    def step(i, carry):
        for b in range(2):
            c = 2 * i + b
            wait_chunk(b)

            @pl.when(c + 2 < NCH)
            def _():
                lab_copy(c + 2).start()

            @pl.when(c >= 2)
            def _():
                for cp in out_copies(c - 2, b):
                    cp.wait()

            compute_chunk(c, b)
            for cp in out_copies(c, b):
                cp.start()

            @pl.when(c + 2 < NCH)
            def _():
                lab_copy(c + 2).wait()
                peel_and_fire(c + 2, b)
        return carry

    lax.fori_loop(0, NCH // 2, step, 0)
    for b in range(2):
        for cp in out_copies(NCH - 2 + b, b):
            cp.wait()


@jax.jit
def _run(x, label, emb1, emb2):
    mesh = plsc.VectorSubcoreMesh(core_axis_name="c", subcore_axis_name="s",
                                  num_cores=NC, num_subcores=NS)
    f = pl.kernel(
        _body,
        out_type=(jax.ShapeDtypeStruct((N * DEPTH,), jnp.float32),
                  jax.ShapeDtypeStruct((N * DEPTH,), jnp.float32)),
        mesh=mesh,
        compiler_params=pltpu.CompilerParams(needs_layout_passes=False,
                                             use_tc_tiling_on_sc=False),
        scratch_types=[
            pltpu.VMEM((TOK_W,), jnp.int32),
            pltpu.VMEM((TOK_W, DIM), jnp.float32),
            pltpu.VMEM((2 * CH,), jnp.int32),
            pltpu.VMEM((4 * T, 1, 2, DEPTH), jnp.int32),
            pltpu.VMEM((2 * CH, DIM), jnp.float32),
            pltpu.VMEM((2 * CH,), jnp.float32),
            pltpu.VMEM((2 * CH,), jnp.float32),
            pltpu.SemaphoreType.DMA,
            pltpu.SemaphoreType.DMA,
            pltpu.SemaphoreType.DMA,
            pltpu.SemaphoreType.DMA,
            pltpu.SemaphoreType.DMA,
            pltpu.SemaphoreType.DMA,
        ],
    )
    return f(x, label, emb1, emb2)


def kernel(inputs, label, emb1, emb2):
    n, p, _, depth = label.shape
    assert (n, p, depth) == (N, 1, DEPTH) and emb1.shape[1] == DIM
    assert label.dtype == jnp.int32 and inputs.dtype == jnp.int32
    o, t = _run(inputs, label, emb1, emb2)
    return o.reshape(N, 1, DEPTH), t.reshape(N, 1, DEPTH)


# speedup vs baseline: 5.1857x; 1.0012x over previous
import jax
import jax.numpy as jnp
from jax import lax
from jax.experimental import pallas as pl
from jax.experimental.pallas import tpu as pltpu
from jax.experimental.pallas import tpu_sc as plsc

N = 16384
DIM = 32
DEPTH = 20
NC = 2
NS = 16
NW = NC * NS
TOK_W = N // NW
T = 64
NCH = TOK_W // T
CH = T * DEPTH
GB = 640
L = 16


def _body(x_hbm, lab_hbm, emb1_hbm, emb2_hbm,
          out_hbm, tgt_hbm,
          idx_v, proj_v, dir_v, lab_v, hvec_v, out_v, tgt_v,
          sem_p, sem_l, sem_g0, sem_g1, sem_o0, sem_o1):
    sem_g = (sem_g0, sem_g1)
    sem_o = (sem_o0, sem_o1)
    wid = lax.axis_index("s") * NC + lax.axis_index("c")
    tok0 = wid * TOK_W
    iota = lax.iota(jnp.int32, L)
    zeros = jnp.zeros((L,), jnp.int32)
    ones = jnp.full((L,), 1, jnp.int32)

    pltpu.sync_copy(x_hbm.at[pl.ds(tok0, TOK_W)], idx_v)
    for j in range(TOK_W // GB):
        pltpu.async_copy(
            emb1_hbm.at[idx_v.at[pl.ds(j * GB, GB)]],
            proj_v.at[pl.ds(j * GB, GB)], sem_p)

    def lab_copy(c):
        return pltpu.make_async_copy(
            lab_hbm.at[pl.ds(tok0 + c * T, T)],
            lab_v.at[pl.ds((c & 3) * T, T)], sem_l)

    def peel_and_fire(c, b):
        for blk in range(T // L):
            tok = (c & 3) * T + blk * L + iota
            dst = b * CH + blk * L * DEPTH + iota * DEPTH
            for k in range(DEPTH):
                v = plsc.load_gather(lab_v, [tok, zeros, zeros,
                                             jnp.full((L,), k, jnp.int32)])
                plsc.store_scatter(dir_v, [dst + k], v)
        for j in range(CH // GB):
            pltpu.async_copy(
                emb2_hbm.at[dir_v.at[pl.ds(b * CH + j * GB, GB)]],
                hvec_v.at[pl.ds(b * CH + j * GB, GB)], sem_g[b])

    def wait_chunk(b):
        for j in range(CH // GB):
            pltpu.make_async_copy(
                emb2_hbm.at[dir_v.at[pl.ds(b * CH + j * GB, GB)]],
                hvec_v.at[pl.ds(b * CH + j * GB, GB)], sem_g[b]).wait()

    for b in range(2):
        lab_copy(jnp.int32(b)).start()
        lab_copy(jnp.int32(b)).wait()
        peel_and_fire(jnp.int32(b), b)
    for j in range(TOK_W // GB):
        pltpu.make_async_copy(
            emb1_hbm.at[idx_v.at[pl.ds(j * GB, GB)]],
            proj_v.at[pl.ds(j * GB, GB)], sem_p).wait()

    def out_copies(c, b):
        return (
            pltpu.make_async_copy(
                out_v.at[pl.ds(b * CH, CH)],
                out_hbm.at[pl.ds(tok0 * DEPTH + c * CH, CH)], sem_o[b]),
            pltpu.make_async_copy(
                tgt_v.at[pl.ds(b * CH, CH)],
                tgt_hbm.at[pl.ds(tok0 * DEPTH + c * CH, CH)], sem_o[b]),
        )

    def compute_chunk(c, b):
        for blk in range(T // L):
            rows = [b * CH + blk * L * DEPTH + iota * DEPTH + k
                    for k in range(DEPTH)]
            tok = (c & 3) * T + blk * L + iota
            rowp = c * T + blk * L + iota

            def dstep(d, accs):
                dv = jnp.full((L,), d, jnp.int32)
                pv = plsc.load_gather(proj_v, [rowp, dv])
                return tuple(
                    accs[k] + pv * plsc.load_gather(hvec_v, [rows[k], dv])
                    for k in range(DEPTH))

            accs = lax.fori_loop(
                0, DIM, dstep, (jnp.zeros((L,), jnp.float32),) * DEPTH)
            for k in range(DEPTH):
                x = accs[k]
                o = 1.0 / (1.0 + jnp.exp(-x))
                labv = plsc.load_gather(lab_v, [tok, zeros, ones,
                                                jnp.full((L,), k, jnp.int32)])
                mi = jnp.where(x >= 0.0, 1, 0)
                t = jnp.where(mi == labv, 1.0, 0.0)
                plsc.store_scatter(out_v, [rows[k]], o)
                plsc.store_scatter(tgt_v, [rows[k]], t)

    def step(i, carry):
        for b in range(2):
            c = 2 * i + b
            wait_chunk(b)

            @pl.when(c + 2 < NCH)
            def _():
                lab_copy(c + 2).start()

            @pl.when(c >= 2)
            def _():
                for cp in out_copies(c - 2, b):
                    cp.wait()

            compute_chunk(c, b)
            for cp in out_copies(c, b):
                cp.start()

            @pl.when(c + 2 < NCH)
            def _():
                lab_copy(c + 2).wait()
                peel_and_fire(c + 2, b)
        return carry

    lax.fori_loop(0, NCH // 2, step, 0)
    for b in range(2):
        for cp in out_copies(NCH - 2 + b, b):
            cp.wait()


@jax.jit
def _run(x, label, emb1, emb2):
    mesh = plsc.VectorSubcoreMesh(core_axis_name="c", subcore_axis_name="s",
                                  num_cores=NC, num_subcores=NS)
    f = pl.kernel(
        _body,
        out_type=(jax.ShapeDtypeStruct((N * DEPTH,), jnp.float32),
                  jax.ShapeDtypeStruct((N * DEPTH,), jnp.float32)),
        mesh=mesh,
        compiler_params=pltpu.CompilerParams(needs_layout_passes=False,
                                             use_tc_tiling_on_sc=False),
        scratch_types=[
            pltpu.VMEM((TOK_W,), jnp.int32),
            pltpu.VMEM((TOK_W, DIM), jnp.float32),
            pltpu.VMEM((2 * CH,), jnp.int32),
            pltpu.VMEM((4 * T, 1, 2, DEPTH), jnp.int32),
            pltpu.VMEM((2 * CH, DIM), jnp.float32),
            pltpu.VMEM((2 * CH,), jnp.float32),
            pltpu.VMEM((2 * CH,), jnp.float32),
            pltpu.SemaphoreType.DMA,
            pltpu.SemaphoreType.DMA,
            pltpu.SemaphoreType.DMA,
            pltpu.SemaphoreType.DMA,
            pltpu.SemaphoreType.DMA,
            pltpu.SemaphoreType.DMA,
        ],
    )
    return f(x, label, emb1, emb2)


def kernel(inputs, label, emb1, emb2):
    n, p, _, depth = label.shape
    assert (n, p, depth) == (N, 1, DEPTH) and emb1.shape[1] == DIM
    assert label.dtype == jnp.int32 and inputs.dtype == jnp.int32
    o, t = _run(inputs, label, emb1, emb2)
    return o.reshape(N, 1, DEPTH), t.reshape(N, 1, DEPTH)
